# baseline (speedup 1.0000x reference)
import functools

import jax
import jax.numpy as jnp
from jax import lax
from jax.experimental import pallas as pl
from jax.experimental.pallas import tpu as pltpu
from jax.experimental.pallas import tpu_sc as plsc

N_NODES_K = 10000
D_K = 128
N_EDGES_K = 320000

CHUNK = 64
BLOCK_EDGES = 2048
BLOCK_ROWS = BLOCK_EDGES // 128
CHUNKS_PER_BLOCK = BLOCK_EDGES // CHUNK
N_BLOCKS = 160
EDGES_PAD = N_BLOCKS * BLOCK_EDGES
SPLIT_N0 = 9
SPLIT_N1 = 1
ROWS_PAD = 10112
ROWS_PER_TILE = ROWS_PAD // 16
DEG_PAD = 10240
DEG_PER_TILE = DEG_PAD // 16


def _sc_body(x_hbm, pk_hbm, acc_hbm, deg_hbm,
             acc_sh, deg_sh, pk_v, bufs, sidx, didx, ones_v, zbuf, sems):
    cid = lax.axis_index("c")
    sid = lax.axis_index("s")
    row0 = sid * ROWS_PER_TILE

    zeros16 = jnp.zeros((16,), jnp.float32)
    ones16 = jnp.ones((16,), jnp.float32)

    buf0 = bufs[0]

    @pl.loop(0, CHUNK)
    def _zero_rows(r):
        for k in range(8):
            buf0[r, pl.ds(k * 16, 16)] = zeros16

    for k in range(CHUNK // 16):
        ones_v[pl.ds(k * 16, 16)] = ones16

    @pl.loop(0, DEG_PER_TILE // 16)
    def _zero_deg(i):
        zbuf[pl.ds(i * 16, 16)] = zeros16

    for t in range(ROWS_PER_TILE // CHUNK):
        pltpu.sync_copy(buf0, acc_sh.at[pl.ds(row0 + t * CHUNK, CHUNK)])
    rem = ROWS_PER_TILE % CHUNK
    if rem:
        pltpu.sync_copy(buf0.at[pl.ds(0, rem)],
                        acc_sh.at[pl.ds(row0 + ROWS_PER_TILE - rem, rem)])
    pltpu.sync_copy(zbuf, deg_sh.at[pl.ds(sid * DEG_PER_TILE, DEG_PER_TILE)])

    plsc.subcore_barrier()

    def unpack(c, s_v, d_v):
        r = c // 2
        cbase = (c % 2) * CHUNK
        for k in range(CHUNK // 16):
            p = pk_v[r, pl.ds(cbase + k * 16, 16)]
            s_v[pl.ds(k * 16, 16)] = p & 0xFFFF
            d_v[pl.ds(k * 16, 16)] = p >> 16

    def start_gather(s_v, buf, sem):
        pltpu.async_copy(x_hbm.at[s_v], buf, sem)

    def wait_gather(buf, sem):
        pltpu.make_async_copy(x_hbm.at[sidx[0]], buf, sem).wait()

    def do_scatter(d_v, buf):
        pltpu.sync_copy(buf, acc_sh.at[d_v], add=True)
        pltpu.sync_copy(ones_v, deg_sh.at[d_v], add=True)

    nb = jnp.where(cid == 0, SPLIT_N0, SPLIT_N1)
    b0 = jnp.where(cid == 0, sid * SPLIT_N0, 16 * SPLIT_N0 + sid * SPLIT_N1)

    @pl.loop(0, nb)
    def _blocks(t):
        pltpu.sync_copy(pk_hbm.at[b0 + t], pk_v)
        for b in range(2):
            unpack(b, sidx[b], didx[b])
            start_gather(sidx[b], bufs[b], sems[b])

        @pl.loop(0, CHUNKS_PER_BLOCK // 2 - 1)
        def _chunks(i):
            for b in range(2):
                wait_gather(bufs[b], sems[b])
                do_scatter(didx[b], bufs[b])
                unpack(2 * i + 2 + b, sidx[b], didx[b])
                start_gather(sidx[b], bufs[b], sems[b])

        for b in range(2):
            wait_gather(bufs[b], sems[b])
            do_scatter(didx[b], bufs[b])

    plsc.subcore_barrier()

    for t in range(ROWS_PER_TILE // CHUNK):
        pltpu.sync_copy(acc_sh.at[pl.ds(row0 + t * CHUNK, CHUNK)],
                        acc_hbm.at[cid, pl.ds(row0 + t * CHUNK, CHUNK)])
    if rem:
        pltpu.sync_copy(acc_sh.at[pl.ds(row0 + ROWS_PER_TILE - rem, rem)],
                        acc_hbm.at[cid, pl.ds(row0 + ROWS_PER_TILE - rem, rem)])
    pltpu.sync_copy(deg_sh.at[pl.ds(sid * DEG_PER_TILE, DEG_PER_TILE)],
                    deg_hbm.at[cid, pl.ds(sid * DEG_PER_TILE, DEG_PER_TILE)])


_sc_aggregate = functools.partial(
    pl.kernel,
    out_type=(
        jax.ShapeDtypeStruct((2, ROWS_PAD, D_K), jnp.float32),
        jax.ShapeDtypeStruct((2, DEG_PAD), jnp.float32),
    ),
    mesh=plsc.VectorSubcoreMesh(core_axis_name="c", subcore_axis_name="s"),
    scratch_types=[
        pltpu.VMEM_SHARED((ROWS_PAD, D_K), jnp.float32),
        pltpu.VMEM_SHARED((DEG_PAD,), jnp.float32),
        pltpu.VMEM((BLOCK_ROWS, 128), jnp.int32),
        tuple(pltpu.VMEM((CHUNK, D_K), jnp.float32) for _ in range(2)),
        tuple(pltpu.VMEM((CHUNK,), jnp.int32) for _ in range(2)),
        tuple(pltpu.VMEM((CHUNK,), jnp.int32) for _ in range(2)),
        pltpu.VMEM((CHUNK,), jnp.float32),
        pltpu.VMEM((DEG_PER_TILE,), jnp.float32),
        tuple(pltpu.SemaphoreType.DMA for _ in range(2)),
    ],
)(_sc_body)


def _tc_body(x_ref, acc_ref, deg_ref, ws_ref, wn_ref, b_ref, o_ref):
    s = acc_ref[0] + acc_ref[1]
    d = deg_ref[0] + deg_ref[1]
    h = s / jnp.maximum(d, 1.0)
    o_ref[...] = (
        jnp.dot(x_ref[...], ws_ref[...], preferred_element_type=jnp.float32,
                precision=lax.Precision.HIGHEST)
        + jnp.dot(h, wn_ref[...], preferred_element_type=jnp.float32,
                  precision=lax.Precision.HIGHEST)
        + b_ref[...]
    )


def _tc_dense(x, acc, deg3, ws_t, wn_t, bias):
    blk = 5000
    grid = (N_NODES_K // blk,)
    return pl.pallas_call(
        _tc_body,
        grid=grid,
        in_specs=[
            pl.BlockSpec((blk, D_K), lambda i: (i, 0)),
            pl.BlockSpec((2, blk, D_K), lambda i: (0, i, 0)),
            pl.BlockSpec((2, blk, 1), lambda i: (0, i, 0)),
            pl.BlockSpec((D_K, D_K), lambda i: (0, 0)),
            pl.BlockSpec((D_K, D_K), lambda i: (0, 0)),
            pl.BlockSpec((1, D_K), lambda i: (0, 0)),
        ],
        out_specs=pl.BlockSpec((blk, D_K), lambda i: (i, 0)),
        out_shape=jax.ShapeDtypeStruct((N_NODES_K, D_K), jnp.float32),
    )(x, acc, deg3, ws_t, wn_t, bias)


@jax.jit
def kernel(x, edge_index, W_self, b_self, W_neigh, b_neigh):
    src = edge_index[0].astype(jnp.int32)
    dst = edge_index[1].astype(jnp.int32)
    pad = EDGES_PAD - N_EDGES_K
    src_p = jnp.concatenate([src, jnp.zeros((pad,), jnp.int32)])
    dst_p = jnp.concatenate([dst, jnp.full((pad,), N_NODES_K, jnp.int32)])
    packed = (src_p | (dst_p << 16)).reshape(N_BLOCKS, BLOCK_ROWS, 128)

    acc, deg = _sc_aggregate(x, packed)

    deg3 = deg.reshape(2, DEG_PAD, 1)
    bias = (b_self + b_neigh).reshape(1, D_K)
    return _tc_dense(x, acc, deg3, W_self.T, W_neigh.T, bias)


# speedup vs baseline: 1.0181x; 1.0181x over previous
import functools

import jax
import jax.numpy as jnp
from jax import lax
from jax.experimental import pallas as pl
from jax.experimental.pallas import tpu as pltpu
from jax.experimental.pallas import tpu_sc as plsc

N_NODES_K = 10000
D_K = 128
N_EDGES_K = 320000

CHUNK = 64
BLOCK_EDGES = 2048
BLOCK_ROWS = BLOCK_EDGES // 128
CHUNKS_PER_BLOCK = BLOCK_EDGES // CHUNK
N_BLOCKS = 160
EDGES_PAD = N_BLOCKS * BLOCK_EDGES
SPLIT_N0 = 9
SPLIT_N1 = 1
ROWS_PAD = 10112
ROWS_PER_TILE = ROWS_PAD // 16
DEG_PAD = 10240
DEG_PER_TILE = DEG_PAD // 16


def _sc_body(x_hbm, pk_hbm, acc_hbm, deg_hbm,
             acc_sh, deg_sh, pk_v, bufs, sidx, didx, ones_v, zbuf, sems):
    cid = lax.axis_index("c")
    sid = lax.axis_index("s")
    row0 = sid * ROWS_PER_TILE

    zeros16 = jnp.zeros((16,), jnp.float32)
    ones16 = jnp.ones((16,), jnp.float32)

    buf0 = bufs[0]

    @pl.loop(0, CHUNK)
    def _zero_rows(r):
        for k in range(8):
            buf0[r, pl.ds(k * 16, 16)] = zeros16

    for k in range(CHUNK // 16):
        ones_v[pl.ds(k * 16, 16)] = ones16

    @pl.loop(0, DEG_PER_TILE // 16)
    def _zero_deg(i):
        zbuf[pl.ds(i * 16, 16)] = zeros16

    for t in range(ROWS_PER_TILE // CHUNK):
        pltpu.sync_copy(buf0, acc_sh.at[pl.ds(row0 + t * CHUNK, CHUNK)])
    rem = ROWS_PER_TILE % CHUNK
    if rem:
        pltpu.sync_copy(buf0.at[pl.ds(0, rem)],
                        acc_sh.at[pl.ds(row0 + ROWS_PER_TILE - rem, rem)])
    pltpu.sync_copy(zbuf, deg_sh.at[pl.ds(sid * DEG_PER_TILE, DEG_PER_TILE)])

    plsc.subcore_barrier()

    def unpack(c, s_v, d_v):
        r = c // 2
        cbase = (c % 2) * CHUNK
        for k in range(CHUNK // 16):
            p = pk_v[r, pl.ds(cbase + k * 16, 16)]
            s_v[pl.ds(k * 16, 16)] = p & 0xFFFF
            d_v[pl.ds(k * 16, 16)] = p >> 16

    def start_gather(s_v, buf, sem):
        pltpu.async_copy(x_hbm.at[s_v], buf, sem)

    def wait_gather(buf, sem):
        pltpu.make_async_copy(x_hbm.at[sidx[0]], buf, sem).wait()

    def do_scatter(d_v, buf):
        pltpu.sync_copy(buf, acc_sh.at[d_v], add=True)
        pltpu.sync_copy(ones_v, deg_sh.at[d_v], add=True)

    nb = jnp.where(cid == 0, SPLIT_N0, SPLIT_N1)
    b0 = jnp.where(cid == 0, sid * SPLIT_N0, 16 * SPLIT_N0 + sid * SPLIT_N1)

    @pl.loop(0, nb)
    def _blocks(t):
        pltpu.sync_copy(pk_hbm.at[b0 + t], pk_v)
        for b in range(2):
            unpack(b, sidx[b], didx[b])
            start_gather(sidx[b], bufs[b], sems[b])

        @pl.loop(0, CHUNKS_PER_BLOCK // 2 - 1)
        def _chunks(i):
            for b in range(2):
                wait_gather(bufs[b], sems[b])
                do_scatter(didx[b], bufs[b])
                unpack(2 * i + 2 + b, sidx[b], didx[b])
                start_gather(sidx[b], bufs[b], sems[b])

        for b in range(2):
            wait_gather(bufs[b], sems[b])
            do_scatter(didx[b], bufs[b])

    plsc.subcore_barrier()

    for t in range(ROWS_PER_TILE // CHUNK):
        pltpu.sync_copy(acc_sh.at[pl.ds(row0 + t * CHUNK, CHUNK)],
                        acc_hbm.at[cid, pl.ds(row0 + t * CHUNK, CHUNK)])
    if rem:
        pltpu.sync_copy(acc_sh.at[pl.ds(row0 + ROWS_PER_TILE - rem, rem)],
                        acc_hbm.at[cid, pl.ds(row0 + ROWS_PER_TILE - rem, rem)])
    pltpu.sync_copy(deg_sh.at[pl.ds(sid * DEG_PER_TILE, DEG_PER_TILE)],
                    deg_hbm.at[cid, pl.ds(sid * DEG_PER_TILE, DEG_PER_TILE)])


_sc_aggregate = functools.partial(
    pl.kernel,
    out_type=(
        jax.ShapeDtypeStruct((2, ROWS_PAD, D_K), jnp.float32),
        jax.ShapeDtypeStruct((2, DEG_PAD), jnp.float32),
    ),
    mesh=plsc.VectorSubcoreMesh(core_axis_name="c", subcore_axis_name="s"),
    scratch_types=[
        pltpu.VMEM_SHARED((ROWS_PAD, D_K), jnp.float32),
        pltpu.VMEM_SHARED((DEG_PAD,), jnp.float32),
        pltpu.VMEM((BLOCK_ROWS, 128), jnp.int32),
        tuple(pltpu.VMEM((CHUNK, D_K), jnp.float32) for _ in range(2)),
        tuple(pltpu.VMEM((CHUNK,), jnp.int32) for _ in range(2)),
        tuple(pltpu.VMEM((CHUNK,), jnp.int32) for _ in range(2)),
        pltpu.VMEM((CHUNK,), jnp.float32),
        pltpu.VMEM((DEG_PER_TILE,), jnp.float32),
        tuple(pltpu.SemaphoreType.DMA for _ in range(2)),
    ],
)(_sc_body)


def _tc_body(x_ref, acc_ref, deg_ref, ws_ref, wn_ref, b_ref, o_ref):
    s = acc_ref[0] + acc_ref[1]
    d = deg_ref[0] + deg_ref[1]
    h = s / jnp.maximum(d, 1.0)
    o_ref[...] = (
        jnp.dot(x_ref[...], ws_ref[...], preferred_element_type=jnp.float32,
                precision=lax.Precision.HIGHEST)
        + jnp.dot(h, wn_ref[...], preferred_element_type=jnp.float32,
                  precision=lax.Precision.HIGHEST)
        + b_ref[...]
    )


def _tc_dense(x, acc, deg3, ws_t, wn_t, bias):
    blk = 2000
    grid = (N_NODES_K // blk,)
    return pl.pallas_call(
        _tc_body,
        grid=grid,
        in_specs=[
            pl.BlockSpec((blk, D_K), lambda i: (i, 0)),
            pl.BlockSpec((2, blk, D_K), lambda i: (0, i, 0)),
            pl.BlockSpec((2, blk, 1), lambda i: (0, i, 0)),
            pl.BlockSpec((D_K, D_K), lambda i: (0, 0)),
            pl.BlockSpec((D_K, D_K), lambda i: (0, 0)),
            pl.BlockSpec((1, D_K), lambda i: (0, 0)),
        ],
        out_specs=pl.BlockSpec((blk, D_K), lambda i: (i, 0)),
        out_shape=jax.ShapeDtypeStruct((N_NODES_K, D_K), jnp.float32),
    )(x, acc, deg3, ws_t, wn_t, bias)


@jax.jit
def kernel(x, edge_index, W_self, b_self, W_neigh, b_neigh):
    src = edge_index[0].astype(jnp.int32)
    dst = edge_index[1].astype(jnp.int32)
    pad = EDGES_PAD - N_EDGES_K
    src_p = jnp.concatenate([src, jnp.zeros((pad,), jnp.int32)])
    dst_p = jnp.concatenate([dst, jnp.full((pad,), N_NODES_K, jnp.int32)])
    packed = (src_p | (dst_p << 16)).reshape(N_BLOCKS, BLOCK_ROWS, 128)

    acc, deg = _sc_aggregate(x, packed)

    deg3 = deg.reshape(2, DEG_PAD, 1)
    bias = (b_self + b_neigh).reshape(1, D_K)
    return _tc_dense(x, acc, deg3, W_self.T, W_neigh.T, bias)
